# baseline (speedup 1.0000x reference)
import jax
import jax.numpy as jnp
from jax import lax
from jax.experimental import pallas as pl
from jax.experimental.pallas import tpu as pltpu
from jax.experimental.pallas import tpu_sc as plsc

N = 10000
H = 128
NC = 2
NS = 16
L = 16
NW = NC * NS
E_RAW = 320000
ETOT = E_RAW + N
CHUNK = 104
NCH = 100
EPT = NCH * CHUNK
EPAD = EPT * NW
NP2 = 10112
NPT = NP2 // NS

_f32 = jnp.float32
_mesh = plsc.VectorSubcoreMesh(
    core_axis_name="c", subcore_axis_name="s", num_cores=NC, num_subcores=NS)
_sc_params = pltpu.CompilerParams(needs_layout_passes=False)


def _leaky(t):
  return jnp.maximum(t, 0.2 * t)


def _attn_rows(a3, hw):
  a2 = lax.dot_general(a3, hw, (((0,), (1,)), ((), ())),
                       preferred_element_type=_f32)
  m = jnp.max(a2[0:1], axis=1, keepdims=True) \
      + jnp.max(a2[1:2], axis=1, keepdims=True)
  krow = jnp.broadcast_to(jnp.maximum(m, 0.0), (1, a2.shape[1]))
  return jnp.concatenate([a2[0:2], krow, a2[3:8]], axis=0)


def _norm_h(acc_ref, den0_ref, den1_ref, bias_ref):
  den = den0_ref[...] + den1_ref[...]
  h = (acc_ref[0, :N] + acc_ref[1, :N]) / den + bias_ref[...]
  return jnp.maximum(h, 0.0)


def _tc0_body(x_ref, wemb_ref, bemb_ref, w1_ref, a3_ref, hw_ref, aa_ref):
  h = jnp.dot(x_ref[...], wemb_ref[...], preferred_element_type=_f32)
  h = h + bemb_ref[...]
  hw = jnp.dot(h, w1_ref[...], preferred_element_type=_f32)
  hw_ref[...] = hw
  aa_ref[...] = _attn_rows(a3_ref[...], hw)


def _tc_layer_body(acc_ref, den0_ref, den1_ref, bias_ref, w_ref, a3_ref,
                   hw_ref, aa_ref):
  h = _norm_h(acc_ref, den0_ref, den1_ref, bias_ref)
  hw = jnp.dot(h, w_ref[...], preferred_element_type=_f32)
  hw_ref[...] = hw
  aa_ref[...] = _attn_rows(a3_ref[...], hw)


def _tc_final_body(acc_ref, den0_ref, den1_ref, bias_ref, wo1_ref, bo1_ref,
                   wo2_ref, bo2_ref, y_ref):
  h = _norm_h(acc_ref, den0_ref, den1_ref, bias_ref)
  o1 = jnp.maximum(
      jnp.dot(h, wo1_ref[...], preferred_element_type=_f32) + bo1_ref[...],
      0.0)
  y_ref[...] = jnp.dot(o1, wo2_ref[...], preferred_element_type=_f32) \
      + bo2_ref[...]


def _tc0(x, wemb, bemb, w1, a3):
  return pl.pallas_call(
      _tc0_body,
      out_shape=(jax.ShapeDtypeStruct((N, H), _f32),
                 jax.ShapeDtypeStruct((8, N), _f32)),
  )(x, wemb, bemb, w1, a3)


def _tc_layer(acc, den0, den1, bias, w, a3):
  return pl.pallas_call(
      _tc_layer_body,
      out_shape=(jax.ShapeDtypeStruct((N, H), _f32),
                 jax.ShapeDtypeStruct((8, N), _f32)),
  )(acc, den0, den1, bias, w, a3)


def _tc_final(acc, den0, den1, bias, wo1, bo1, wo2, bo2):
  return pl.pallas_call(
      _tc_final_body,
      out_shape=jax.ShapeDtypeStruct((N, H), _f32),
  )(acc, den0, den1, bias, wo1, bo1, wo2, bo2)


def _sc_gat_body(sd, a_s, a_d, kvec, zeros_n, zeros_nh, hw,
                 den0, den1, acc_out,
                 a_s_v, a_d_v, kv,
                 sd0, ex0, rows0, sd1, ex1, rows1,
                 den_sh, acc_sh, gsem, ssem, dsem):
  cid = lax.axis_index("c")
  sid = lax.axis_index("s")
  wid = sid * NC + cid

  @pl.when(sid == 0)
  def _():
    pltpu.sync_copy(zeros_n, den_sh)

  pltpu.sync_copy(zeros_nh.at[pl.ds(sid * NPT, NPT)],
                  acc_sh.at[pl.ds(sid * NPT, NPT)])
  pltpu.sync_copy(a_s, a_s_v)
  pltpu.sync_copy(a_d, a_d_v)
  pltpu.sync_copy(kvec, kv)
  plsc.subcore_barrier()

  kvv = kv[...]
  gbase = wid * NCH
  base = wid * EPT
  bufs = ((sd0, ex0, rows0), (sd1, ex1, rows1))

  pltpu.sync_copy(sd.at[gbase], sd0)
  pltpu.async_copy(hw.at[sd0.at[0]], rows0, gsem)

  def pair_body(i, _):
    for b in (0, 1):
      sdb, exb, rowsb = bufs[b]
      sdn, exn, rowsn = bufs[1 - b]
      c2 = 2 * i + b
      off = base + c2 * CHUNK

      @pl.when(c2 + 1 < NCH)
      def _():
        @pl.when(c2 >= 1)
        def _():
          pltpu.make_async_copy(rowsn, acc_sh.at[sdn.at[1]], ssem).wait()
          pltpu.make_async_copy(exn, den_sh.at[sdn.at[1]], dsem).wait()
        pltpu.sync_copy(sd.at[gbase + c2 + 1], sdn)
        pltpu.async_copy(hw.at[sdn.at[0]], rowsn, gsem)

      def vec_body(k, _):
        s16 = sdb[0, pl.ds(k * L, L)]
        d16 = sdb[1, pl.ds(k * L, L)]
        asv = plsc.load_gather(a_s_v, [s16])
        adv = plsc.load_gather(a_d_v, [d16])
        ex = jnp.exp(_leaky(asv + adv) - kvv)
        gid = off + k * L + lax.iota(jnp.int32, L)
        exb[pl.ds(k * L, L)] = jnp.where(gid < ETOT, ex, 0.0)
        return 0

      lax.fori_loop(0, CHUNK // L, vec_body, 0)
      pltpu.async_copy(exb, den_sh.at[sdb.at[1]], dsem, add=True)
      pltpu.make_async_copy(hw.at[sdb.at[0]], rowsb, gsem).wait()

      def row_body(g, _):
        a16 = exb[pl.ds(g * L, L)]
        for rr in range(L):
          r = g * L + rr
          a = a16[rr]
          for j in range(H // L):
            rowsb[r, pl.ds(j * L, L)] = rowsb[r, pl.ds(j * L, L)] * a
        return 0

      lax.fori_loop(0, CHUNK // L, row_body, 0)
      pltpu.async_copy(rowsb, acc_sh.at[sdb.at[1]], ssem, add=True)
    return 0

  lax.fori_loop(0, NCH // 2, pair_body, 0)

  for b in (0, 1):
    sdb, exb, rowsb = bufs[b]
    pltpu.make_async_copy(rowsb, acc_sh.at[sdb.at[1]], ssem).wait()
    pltpu.make_async_copy(exb, den_sh.at[sdb.at[1]], dsem).wait()

  plsc.subcore_barrier()

  @pl.when((sid == 0) & (cid == 0))
  def _():
    pltpu.sync_copy(den_sh, den0)

  @pl.when((sid == 0) & (cid == 1))
  def _():
    pltpu.sync_copy(den_sh, den1)

  pltpu.sync_copy(acc_sh.at[pl.ds(sid * NPT, NPT)],
                  acc_out.at[cid, pl.ds(sid * NPT, NPT)])


def _sc_gat(sd, a_s, a_d, kvec, zeros_n, zeros_nh, hw):
  return pl.kernel(
      _sc_gat_body,
      out_type=(jax.ShapeDtypeStruct((N,), _f32),
                jax.ShapeDtypeStruct((N,), _f32),
                jax.ShapeDtypeStruct((NC, NP2, H), _f32)),
      mesh=_mesh,
      compiler_params=_sc_params,
      scratch_types=[
          pltpu.VMEM((N,), _f32),
          pltpu.VMEM((N,), _f32),
          pltpu.VMEM((L,), _f32),
          pltpu.VMEM((2, CHUNK), jnp.int32),
          pltpu.VMEM((CHUNK,), _f32),
          pltpu.VMEM((CHUNK, H), _f32),
          pltpu.VMEM((2, CHUNK), jnp.int32),
          pltpu.VMEM((CHUNK,), _f32),
          pltpu.VMEM((CHUNK, H), _f32),
          pltpu.VMEM_SHARED((N,), _f32),
          pltpu.VMEM_SHARED((NP2, H), _f32),
          pltpu.SemaphoreType.DMA,
          pltpu.SemaphoreType.DMA,
          pltpu.SemaphoreType.DMA,
      ],
  )(sd, a_s, a_d, kvec, zeros_n, zeros_nh, hw)


def _pad_a3(asrc, adst):
  a3 = jnp.stack([asrc, adst], axis=1)
  return jnp.pad(a3, ((0, 0), (0, 6)))


def _gat_aggregate(sd, aa, hw, zeros_n, zeros_nh):
  a_s, a_d, kvec = aa[0], aa[1], aa[2, :L]
  den0, den1, acc = _sc_gat(sd, a_s, a_d, kvec, zeros_n, zeros_nh, hw)
  return acc, den0[:, None], den1[:, None]


def kernel(x, edge_index, edge_attr, W_emb, b_emb,
           W1, asrc1, adst1, bias1, W2, asrc2, adst2, bias2,
           W3, asrc3, adst3, bias3, Wo1, bo1, Wo2, bo2):
  del edge_attr
  loops = jnp.arange(N, dtype=edge_index.dtype)
  pad = EPAD - ETOT
  srcr = jnp.pad(jnp.concatenate([edge_index[0], loops]), (0, pad))
  dstr = jnp.pad(jnp.concatenate([edge_index[1], loops]), (0, pad))
  sd = jnp.stack([srcr.reshape(-1, CHUNK), dstr.reshape(-1, CHUNK)], axis=1)
  zeros_n = jnp.zeros((N,), _f32)
  zeros_nh = jnp.zeros((NP2, H), _f32)

  hw, aa = _tc0(x, W_emb, b_emb, W1, _pad_a3(asrc1, adst1))
  layers = ((bias1, W2, asrc2, adst2), (bias2, W3, asrc3, adst3))
  for bias, w_next, a_next, d_next in layers:
    acc, den0, den1 = _gat_aggregate(sd, aa, hw, zeros_n, zeros_nh)
    hw, aa = _tc_layer(acc, den0, den1, bias, w_next, _pad_a3(a_next, d_next))
  acc, den0, den1 = _gat_aggregate(sd, aa, hw, zeros_n, zeros_nh)
  return _tc_final(acc, den0, den1, bias3, Wo1, bo1, Wo2, bo2)


# speedup vs baseline: 1.1711x; 1.1711x over previous
import jax
import jax.numpy as jnp
from jax import lax
from jax.experimental import pallas as pl
from jax.experimental.pallas import tpu as pltpu
from jax.experimental.pallas import tpu_sc as plsc

N = 10000
H = 128
NC = 2
NS = 16
L = 16
NW = NC * NS
E_RAW = 320000
ETOT = E_RAW + N
CHUNK = 96
NCH = 108
EPT = NCH * CHUNK
EPAD = EPT * NW
NP2 = 10112
NPT = NP2 // NS

_f32 = jnp.float32
_mesh = plsc.VectorSubcoreMesh(
    core_axis_name="c", subcore_axis_name="s", num_cores=NC, num_subcores=NS)
_sc_params = pltpu.CompilerParams(needs_layout_passes=False)


def _leaky(t):
  return jnp.maximum(t, 0.2 * t)


def _attn_rows(a3, hw):
  a2 = lax.dot_general(a3, hw, (((0,), (1,)), ((), ())),
                       preferred_element_type=_f32)
  m = jnp.max(a2[0:1], axis=1, keepdims=True) \
      + jnp.max(a2[1:2], axis=1, keepdims=True)
  krow = jnp.broadcast_to(jnp.maximum(m, 0.0), (1, a2.shape[1]))
  return jnp.concatenate([a2[0:2], krow, a2[3:8]], axis=0)


def _norm_h(acc_ref, den0_ref, den1_ref, bias_ref):
  den = den0_ref[...] + den1_ref[...]
  h = (acc_ref[0, :N] + acc_ref[1, :N]) / den + bias_ref[...]
  return jnp.maximum(h, 0.0)


def _tc0_body(x_ref, wemb_ref, bemb_ref, w1_ref, a3_ref, hw_ref, aa_ref):
  h = jnp.dot(x_ref[...], wemb_ref[...], preferred_element_type=_f32)
  h = h + bemb_ref[...]
  hw = jnp.dot(h, w1_ref[...], preferred_element_type=_f32)
  hw_ref[...] = hw
  aa_ref[...] = _attn_rows(a3_ref[...], hw)


def _tc_layer_body(acc_ref, den0_ref, den1_ref, bias_ref, w_ref, a3_ref,
                   hw_ref, aa_ref):
  h = _norm_h(acc_ref, den0_ref, den1_ref, bias_ref)
  hw = jnp.dot(h, w_ref[...], preferred_element_type=_f32)
  hw_ref[...] = hw
  aa_ref[...] = _attn_rows(a3_ref[...], hw)


def _tc_final_body(acc_ref, den0_ref, den1_ref, bias_ref, wo1_ref, bo1_ref,
                   wo2_ref, bo2_ref, y_ref):
  h = _norm_h(acc_ref, den0_ref, den1_ref, bias_ref)
  o1 = jnp.maximum(
      jnp.dot(h, wo1_ref[...], preferred_element_type=_f32) + bo1_ref[...],
      0.0)
  y_ref[...] = jnp.dot(o1, wo2_ref[...], preferred_element_type=_f32) \
      + bo2_ref[...]


def _tc0(x, wemb, bemb, w1, a3):
  return pl.pallas_call(
      _tc0_body,
      out_shape=(jax.ShapeDtypeStruct((N, H), _f32),
                 jax.ShapeDtypeStruct((8, N), _f32)),
  )(x, wemb, bemb, w1, a3)


def _tc_layer(acc, den0, den1, bias, w, a3):
  return pl.pallas_call(
      _tc_layer_body,
      out_shape=(jax.ShapeDtypeStruct((N, H), _f32),
                 jax.ShapeDtypeStruct((8, N), _f32)),
  )(acc, den0, den1, bias, w, a3)


def _tc_final(acc, den0, den1, bias, wo1, bo1, wo2, bo2):
  return pl.pallas_call(
      _tc_final_body,
      out_shape=jax.ShapeDtypeStruct((N, H), _f32),
  )(acc, den0, den1, bias, wo1, bo1, wo2, bo2)


def _sc_gat_body(sd, a_s, a_d, kvec, zeros_n, zeros_nh, hw,
                 den0, den1, acc_out,
                 a_s_v, a_d_v, kv,
                 sd0, ex0, rows0, sd1, ex1, rows1,
                 den_sh, acc_sh, gsem, ssem, dsem):
  cid = lax.axis_index("c")
  sid = lax.axis_index("s")
  wid = sid * NC + cid

  @pl.when(sid == 0)
  def _():
    pltpu.sync_copy(zeros_n, den_sh)

  pltpu.sync_copy(zeros_nh.at[pl.ds(sid * NPT, NPT)],
                  acc_sh.at[pl.ds(sid * NPT, NPT)])
  pltpu.sync_copy(a_s, a_s_v)
  pltpu.sync_copy(a_d, a_d_v)
  pltpu.sync_copy(kvec, kv)
  plsc.subcore_barrier()

  kvv = kv[...]
  gbase = wid * NCH
  base = wid * EPT
  bufs = ((sd0, ex0, rows0), (sd1, ex1, rows1))

  pltpu.sync_copy(sd.at[gbase], sd0)
  pltpu.async_copy(hw.at[sd0.at[0]], rows0, gsem)

  def pair_body(i, _):
    for b in (0, 1):
      sdb, exb, rowsb = bufs[b]
      sdn, exn, rowsn = bufs[1 - b]
      c2 = 2 * i + b
      off = base + c2 * CHUNK

      @pl.when(c2 + 1 < NCH)
      def _():
        @pl.when(c2 >= 1)
        def _():
          pltpu.make_async_copy(rowsn, acc_sh.at[sdn.at[1]], ssem).wait()
          pltpu.make_async_copy(exn, den_sh.at[sdn.at[1]], dsem).wait()
        pltpu.sync_copy(sd.at[gbase + c2 + 1], sdn)
        pltpu.async_copy(hw.at[sdn.at[0]], rowsn, gsem)

      def vec_body(k, _):
        s16 = sdb[0, pl.ds(k * L, L)]
        d16 = sdb[1, pl.ds(k * L, L)]
        asv = plsc.load_gather(a_s_v, [s16])
        adv = plsc.load_gather(a_d_v, [d16])
        ex = jnp.exp(_leaky(asv + adv) - kvv)
        gid = off + k * L + lax.iota(jnp.int32, L)
        exb[pl.ds(k * L, L)] = jnp.where(gid < ETOT, ex, 0.0)
        return 0

      lax.fori_loop(0, CHUNK // L, vec_body, 0)
      pltpu.async_copy(exb, den_sh.at[sdb.at[1]], dsem, add=True)
      pltpu.make_async_copy(hw.at[sdb.at[0]], rowsb, gsem).wait()

      def row_body(g, _):
        a16 = exb[pl.ds(g * L, L)]
        for rr in range(L):
          r = g * L + rr
          a = a16[rr]
          for j in range(H // L):
            rowsb[r, pl.ds(j * L, L)] = rowsb[r, pl.ds(j * L, L)] * a
        return 0

      lax.fori_loop(0, CHUNK // L, row_body, 0)
      pltpu.async_copy(rowsb, acc_sh.at[sdb.at[1]], ssem, add=True)
    return 0

  lax.fori_loop(0, NCH // 2, pair_body, 0)

  for b in (0, 1):
    sdb, exb, rowsb = bufs[b]
    pltpu.make_async_copy(rowsb, acc_sh.at[sdb.at[1]], ssem).wait()
    pltpu.make_async_copy(exb, den_sh.at[sdb.at[1]], dsem).wait()

  plsc.subcore_barrier()

  @pl.when((sid == 0) & (cid == 0))
  def _():
    pltpu.sync_copy(den_sh, den0)

  @pl.when((sid == 0) & (cid == 1))
  def _():
    pltpu.sync_copy(den_sh, den1)

  pltpu.sync_copy(acc_sh.at[pl.ds(sid * NPT, NPT)],
                  acc_out.at[cid, pl.ds(sid * NPT, NPT)])


def _sc_gat(sd, a_s, a_d, kvec, zeros_n, zeros_nh, hw):
  return pl.kernel(
      _sc_gat_body,
      out_type=(jax.ShapeDtypeStruct((N,), _f32),
                jax.ShapeDtypeStruct((N,), _f32),
                jax.ShapeDtypeStruct((NC, NP2, H), _f32)),
      mesh=_mesh,
      compiler_params=_sc_params,
      scratch_types=[
          pltpu.VMEM((N,), _f32),
          pltpu.VMEM((N,), _f32),
          pltpu.VMEM((L,), _f32),
          pltpu.VMEM((2, CHUNK), jnp.int32),
          pltpu.VMEM((CHUNK,), _f32),
          pltpu.VMEM((CHUNK, H), _f32),
          pltpu.VMEM((2, CHUNK), jnp.int32),
          pltpu.VMEM((CHUNK,), _f32),
          pltpu.VMEM((CHUNK, H), _f32),
          pltpu.VMEM_SHARED((N,), _f32),
          pltpu.VMEM_SHARED((NP2, H), _f32),
          pltpu.SemaphoreType.DMA,
          pltpu.SemaphoreType.DMA,
          pltpu.SemaphoreType.DMA,
      ],
  )(sd, a_s, a_d, kvec, zeros_n, zeros_nh, hw)


def _pad_a3(asrc, adst):
  a3 = jnp.stack([asrc, adst], axis=1)
  return jnp.pad(a3, ((0, 0), (0, 6)))


def _gat_aggregate(sd, aa, hw, zeros_n, zeros_nh):
  a_s, a_d, kvec = aa[0], aa[1], aa[2, :L]
  den0, den1, acc = _sc_gat(sd, a_s, a_d, kvec, zeros_n, zeros_nh, hw)
  return acc, den0[:, None], den1[:, None]


def kernel(x, edge_index, edge_attr, W_emb, b_emb,
           W1, asrc1, adst1, bias1, W2, asrc2, adst2, bias2,
           W3, asrc3, adst3, bias3, Wo1, bo1, Wo2, bo2):
  del edge_attr
  loops = jnp.arange(N, dtype=edge_index.dtype)
  pad = EPAD - ETOT
  srcr = jnp.pad(jnp.concatenate([edge_index[0], loops]), (0, pad))
  dstr = jnp.pad(jnp.concatenate([edge_index[1], loops]), (0, pad))
  sd = jnp.stack([srcr.reshape(-1, CHUNK), dstr.reshape(-1, CHUNK)], axis=1)
  zeros_n = jnp.zeros((N,), _f32)
  zeros_nh = jnp.zeros((NP2, H), _f32)

  hw, aa = _tc0(x, W_emb, b_emb, W1, _pad_a3(asrc1, adst1))
  layers = ((bias1, W2, asrc2, adst2), (bias2, W3, asrc3, adst3))
  for bias, w_next, a_next, d_next in layers:
    acc, den0, den1 = _gat_aggregate(sd, aa, hw, zeros_n, zeros_nh)
    hw, aa = _tc_layer(acc, den0, den1, bias, w_next, _pad_a3(a_next, d_next))
  acc, den0, den1 = _gat_aggregate(sd, aa, hw, zeros_n, zeros_nh)
  return _tc_final(acc, den0, den1, bias3, Wo1, bo1, Wo2, bo2)


# speedup vs baseline: 1.1771x; 1.0051x over previous
import jax
import jax.numpy as jnp
from jax import lax
from jax.experimental import pallas as pl
from jax.experimental.pallas import tpu as pltpu
from jax.experimental.pallas import tpu_sc as plsc

N = 10000
H = 128
NC = 2
NS = 16
L = 16
NW = NC * NS
E_RAW = 320000
ETOT = E_RAW + N
CHUNK = 96
NCH = 108
EPT = NCH * CHUNK
EPAD = EPT * NW
NP2 = 10112
NPT = NP2 // NS

_f32 = jnp.float32
_sc_params = pltpu.CompilerParams(needs_layout_passes=False)


def _mesh():
  return plsc.VectorSubcoreMesh(
      core_axis_name="c", subcore_axis_name="s",
      num_cores=NC, num_subcores=NS)


def _leaky(t):
  return jnp.maximum(t, 0.2 * t)


def _attn_rows(a3, hw):
  a2 = lax.dot_general(a3, hw, (((0,), (1,)), ((), ())),
                       preferred_element_type=_f32)
  m = jnp.max(a2[0:1], axis=1, keepdims=True) \
      + jnp.max(a2[1:2], axis=1, keepdims=True)
  krow = jnp.broadcast_to(jnp.maximum(m, 0.0), (1, a2.shape[1]))
  return jnp.concatenate([a2[0:2], krow, a2[3:8]], axis=0)


def _norm_h(acc_ref, den0_ref, den1_ref, bias_ref):
  den = den0_ref[...] + den1_ref[...]
  h = (acc_ref[0, :N] + acc_ref[1, :N]) / den + bias_ref[...]
  return jnp.maximum(h, 0.0)


def _tc0_body(x_ref, wemb_ref, bemb_ref, w1_ref, a3_ref, hw_ref, aa_ref):
  h = jnp.dot(x_ref[...], wemb_ref[...], preferred_element_type=_f32)
  h = h + bemb_ref[...]
  hw = jnp.dot(h, w1_ref[...], preferred_element_type=_f32)
  hw_ref[...] = hw
  aa_ref[...] = _attn_rows(a3_ref[...], hw)


def _tc_layer_body(acc_ref, den0_ref, den1_ref, bias_ref, w_ref, a3_ref,
                   hw_ref, aa_ref):
  h = _norm_h(acc_ref, den0_ref, den1_ref, bias_ref)
  hw = jnp.dot(h, w_ref[...], preferred_element_type=_f32)
  hw_ref[...] = hw
  aa_ref[...] = _attn_rows(a3_ref[...], hw)


def _tc_final_body(acc_ref, den0_ref, den1_ref, bias_ref, wo1_ref, bo1_ref,
                   wo2_ref, bo2_ref, y_ref):
  h = _norm_h(acc_ref, den0_ref, den1_ref, bias_ref)
  o1 = jnp.maximum(
      jnp.dot(h, wo1_ref[...], preferred_element_type=_f32) + bo1_ref[...],
      0.0)
  y_ref[...] = jnp.dot(o1, wo2_ref[...], preferred_element_type=_f32) \
      + bo2_ref[...]


def _tc0(x, wemb, bemb, w1, a3):
  return pl.pallas_call(
      _tc0_body,
      out_shape=(jax.ShapeDtypeStruct((N, H), _f32),
                 jax.ShapeDtypeStruct((8, N), _f32)),
  )(x, wemb, bemb, w1, a3)


def _tc_layer(acc, den0, den1, bias, w, a3):
  return pl.pallas_call(
      _tc_layer_body,
      out_shape=(jax.ShapeDtypeStruct((N, H), _f32),
                 jax.ShapeDtypeStruct((8, N), _f32)),
  )(acc, den0, den1, bias, w, a3)


def _tc_final(acc, den0, den1, bias, wo1, bo1, wo2, bo2):
  return pl.pallas_call(
      _tc_final_body,
      out_shape=jax.ShapeDtypeStruct((N, H), _f32),
  )(acc, den0, den1, bias, wo1, bo1, wo2, bo2)


def _sc_gat_body(sd, a_s, a_d, kvec, zeros_n, zeros_nh, hw,
                 den0, den1, acc_out,
                 a_s_v, a_d_v, kv,
                 sd0, ex0, rows0, sd1, ex1, rows1,
                 den_sh, acc_sh, gsem, ssem, dsem):
  cid = lax.axis_index("c")
  sid = lax.axis_index("s")
  wid = sid * NC + cid

  @pl.when(sid == 0)
  def _():
    pltpu.sync_copy(zeros_n, den_sh)

  pltpu.sync_copy(zeros_nh.at[pl.ds(sid * NPT, NPT)],
                  acc_sh.at[pl.ds(sid * NPT, NPT)])
  pltpu.sync_copy(a_s, a_s_v)
  pltpu.sync_copy(a_d, a_d_v)
  pltpu.sync_copy(kvec, kv)
  plsc.subcore_barrier()

  kvv = kv[...]
  gbase = wid * NCH
  base = wid * EPT
  bufs = ((sd0, ex0, rows0), (sd1, ex1, rows1))

  pltpu.sync_copy(sd.at[gbase], sd0)
  pltpu.async_copy(hw.at[sd0.at[0]], rows0, gsem)

  def pair_body(i, _):
    for b in (0, 1):
      sdb, exb, rowsb = bufs[b]
      sdn, exn, rowsn = bufs[1 - b]
      c2 = 2 * i + b
      off = base + c2 * CHUNK

      @pl.when(c2 + 1 < NCH)
      def _():
        @pl.when(c2 >= 1)
        def _():
          pltpu.make_async_copy(rowsn, acc_sh.at[sdn.at[1]], ssem).wait()
          pltpu.make_async_copy(exn, den_sh.at[sdn.at[1]], dsem).wait()
        pltpu.sync_copy(sd.at[gbase + c2 + 1], sdn)
        pltpu.async_copy(hw.at[sdn.at[0]], rowsn, gsem)

      def vec_body(k, _):
        s16 = sdb[0, pl.ds(k * L, L)]
        d16 = sdb[1, pl.ds(k * L, L)]
        asv = plsc.load_gather(a_s_v, [s16])
        adv = plsc.load_gather(a_d_v, [d16])
        ex = jnp.exp(_leaky(asv + adv) - kvv)
        gid = off + k * L + lax.iota(jnp.int32, L)
        exb[pl.ds(k * L, L)] = jnp.where(gid < ETOT, ex, 0.0)
        return 0

      lax.fori_loop(0, CHUNK // L, vec_body, 0)
      pltpu.async_copy(exb, den_sh.at[sdb.at[1]], dsem, add=True)
      pltpu.make_async_copy(hw.at[sdb.at[0]], rowsb, gsem).wait()

      def row_body(g, _):
        a16 = exb[pl.ds(g * L, L)]
        for rr in range(L):
          r = g * L + rr
          a = a16[rr]
          for j in range(H // L):
            rowsb[r, pl.ds(j * L, L)] = rowsb[r, pl.ds(j * L, L)] * a
        return 0

      lax.fori_loop(0, CHUNK // L, row_body, 0)
      pltpu.async_copy(rowsb, acc_sh.at[sdb.at[1]], ssem, add=True)
    return 0

  lax.fori_loop(0, NCH // 2, pair_body, 0)

  for b in (0, 1):
    sdb, exb, rowsb = bufs[b]
    pltpu.make_async_copy(rowsb, acc_sh.at[sdb.at[1]], ssem).wait()
    pltpu.make_async_copy(exb, den_sh.at[sdb.at[1]], dsem).wait()

  plsc.subcore_barrier()

  @pl.when((sid == 0) & (cid == 0))
  def _():
    pltpu.sync_copy(den_sh, den0)

  @pl.when((sid == 0) & (cid == 1))
  def _():
    pltpu.sync_copy(den_sh, den1)

  pltpu.sync_copy(acc_sh.at[pl.ds(sid * NPT, NPT)],
                  acc_out.at[cid, pl.ds(sid * NPT, NPT)])


def _sc_gat(sd, a_s, a_d, kvec, zeros_n, zeros_nh, hw):
  return pl.kernel(
      _sc_gat_body,
      out_type=(jax.ShapeDtypeStruct((N,), _f32),
                jax.ShapeDtypeStruct((N,), _f32),
                jax.ShapeDtypeStruct((NC, NP2, H), _f32)),
      mesh=_mesh(),
      compiler_params=_sc_params,
      scratch_types=[
          pltpu.VMEM((N,), _f32),
          pltpu.VMEM((N,), _f32),
          pltpu.VMEM((L,), _f32),
          pltpu.VMEM((2, CHUNK), jnp.int32),
          pltpu.VMEM((CHUNK,), _f32),
          pltpu.VMEM((CHUNK, H), _f32),
          pltpu.VMEM((2, CHUNK), jnp.int32),
          pltpu.VMEM((CHUNK,), _f32),
          pltpu.VMEM((CHUNK, H), _f32),
          pltpu.VMEM_SHARED((N,), _f32),
          pltpu.VMEM_SHARED((NP2, H), _f32),
          pltpu.SemaphoreType.DMA,
          pltpu.SemaphoreType.DMA,
          pltpu.SemaphoreType.DMA,
      ],
  )(sd, a_s, a_d, kvec, zeros_n, zeros_nh, hw)


def _pad_a3(asrc, adst):
  a3 = jnp.stack([asrc, adst], axis=1)
  return jnp.pad(a3, ((0, 0), (0, 6)))


def _gat_aggregate(sd, aa, hw, zeros_n, zeros_nh):
  a_s, a_d, kvec = aa[0], aa[1], aa[2, :L]
  den0, den1, acc = _sc_gat(sd, a_s, a_d, kvec, zeros_n, zeros_nh, hw)
  return acc, den0[:, None], den1[:, None]


def kernel(x, edge_index, edge_attr, W_emb, b_emb,
           W1, asrc1, adst1, bias1, W2, asrc2, adst2, bias2,
           W3, asrc3, adst3, bias3, Wo1, bo1, Wo2, bo2):
  del edge_attr
  loops = jnp.arange(N, dtype=edge_index.dtype)
  pad = EPAD - ETOT
  srcr = jnp.pad(jnp.concatenate([edge_index[0], loops]), (0, pad))
  dstr = jnp.pad(jnp.concatenate([edge_index[1], loops]), (0, pad))
  sd = jnp.stack([srcr.reshape(-1, CHUNK), dstr.reshape(-1, CHUNK)], axis=1)
  zeros_n = jnp.zeros((N,), _f32)
  zeros_nh = jnp.zeros((NP2, H), _f32)

  hw, aa = _tc0(x, W_emb, b_emb, W1, _pad_a3(asrc1, adst1))
  layers = ((bias1, W2, asrc2, adst2), (bias2, W3, asrc3, adst3))
  for bias, w_next, a_next, d_next in layers:
    acc, den0, den1 = _gat_aggregate(sd, aa, hw, zeros_n, zeros_nh)
    hw, aa = _tc_layer(acc, den0, den1, bias, w_next, _pad_a3(a_next, d_next))
  acc, den0, den1 = _gat_aggregate(sd, aa, hw, zeros_n, zeros_nh)
  return _tc_final(acc, den0, den1, bias3, Wo1, bo1, Wo2, bo2)
